# baseline (speedup 1.0000x reference)
import functools

import jax
import jax.numpy as jnp
from jax.experimental import pallas as pl
from jax.experimental.pallas import tpu as pltpu

N = 10000
E = 320000
HID = 256
OUT = 256
CHUNK = 512


def _mm_kernel(x_ref, w_ref, o_ref):
    o_ref[...] = jnp.dot(x_ref[...], w_ref[...], preferred_element_type=jnp.float32)


def _mm(x, w):
    return pl.pallas_call(
        _mm_kernel,
        out_shape=jax.ShapeDtypeStruct((x.shape[0], w.shape[1]), jnp.float32),
    )(x, w)


def _agg1_kernel(src_ref, dst_ref, al_ref, h_ref, o_ref):
    step = pl.program_id(0)

    @pl.when(step == 0)
    def _init():
        o_ref[...] = jnp.zeros_like(o_ref)

    head_mask = jax.lax.broadcasted_iota(jnp.int32, (1, 2 * HID), 1) < HID

    def body(jj, carry):
        base = jj * 8
        msgs = []
        for u in range(8):
            j = base + u
            s = src_ref[j]
            a0 = al_ref[j, 0]
            a1 = al_ref[j, 1]
            row = h_ref[pl.ds(s, 1), :]
            mult = jnp.where(head_mask, a0, a1)
            msgs.append(row * mult)
        for u in range(8):
            j = base + u
            dn = dst_ref[j]
            o_ref[pl.ds(dn, 1), :] = o_ref[pl.ds(dn, 1), :] + msgs[u]
        return carry

    jax.lax.fori_loop(0, CHUNK // 8, body, 0)


def _agg1(src_s, dst_s, alpha_s, h):
    return pl.pallas_call(
        _agg1_kernel,
        grid=(E // CHUNK,),
        in_specs=[
            pl.BlockSpec((CHUNK,), lambda i: (i,), memory_space=pltpu.SMEM),
            pl.BlockSpec((CHUNK,), lambda i: (i,), memory_space=pltpu.SMEM),
            pl.BlockSpec((CHUNK, 2), lambda i: (i, 0), memory_space=pltpu.SMEM),
            pl.BlockSpec((N, 2 * HID), lambda i: (0, 0)),
        ],
        out_specs=pl.BlockSpec((N, 2 * HID), lambda i: (0, 0)),
        out_shape=jax.ShapeDtypeStruct((N, 2 * HID), jnp.float32),
    )(src_s, dst_s, alpha_s, h)


def _gat1(x, edge_index, W, att_src, att_dst, bias):
    src = edge_index[0]
    dst = edge_index[1]
    n = x.shape[0]
    h = _mm(x, W)
    h3 = h.reshape(n, 2, HID)
    a_src = jnp.sum(h3 * att_src[None, :, :], axis=-1)
    a_dst = jnp.sum(h3 * att_dst[None, :, :], axis=-1)
    alpha = a_src[src] + a_dst[dst]
    alpha = jax.nn.leaky_relu(alpha, negative_slope=0.2)
    amax = jax.ops.segment_max(alpha, dst, num_segments=n)
    alpha = jnp.exp(alpha - amax[dst])
    denom = jax.ops.segment_sum(alpha, dst, num_segments=n)
    alpha = alpha / (denom[dst] + 1e-16)
    perm = jnp.argsort(dst, stable=True)
    out = _agg1(src[perm], dst[perm], alpha[perm], h)
    return out + bias


def _gat2(x, edge_index, W, att_src, att_dst, bias):
    src = edge_index[0]
    dst = edge_index[1]
    n = x.shape[0]
    h = _mm(x, W).reshape(n, 1, OUT)
    a_src = jnp.sum(h * att_src[None, :, :], axis=-1)
    a_dst = jnp.sum(h * att_dst[None, :, :], axis=-1)
    alpha = a_src[src] + a_dst[dst]
    alpha = jax.nn.leaky_relu(alpha, negative_slope=0.2)
    amax = jax.ops.segment_max(alpha, dst, num_segments=n)
    alpha = jnp.exp(alpha - amax[dst])
    denom = jax.ops.segment_sum(alpha, dst, num_segments=n)
    alpha = alpha / (denom[dst] + 1e-16)
    msg = h[src] * alpha[:, :, None]
    out = jax.ops.segment_sum(msg, dst, num_segments=n)
    out = out.mean(axis=1)
    return out + bias


def _batchnorm(x, gamma, beta, eps=1e-5):
    mean = x.mean(axis=0)
    var = x.var(axis=0)
    return (x - mean) / jnp.sqrt(var + eps) * gamma + beta


def kernel(x, edge_index, W1, att_src1, att_dst1, b1, bn1_g, bn1_b,
           W2, att_src2, att_dst2, b2, bn2_g, bn2_b):
    h = _gat1(x, edge_index, W1, att_src1, att_dst1, b1)
    h = _batchnorm(h, bn1_g, bn1_b)
    h = jax.nn.relu(h)
    h = _gat2(h, edge_index, W2, att_src2, att_dst2, b2)
    h = _batchnorm(h, bn2_g, bn2_b)
    return h.mean(axis=0, keepdims=True)


# speedup vs baseline: 1.0010x; 1.0010x over previous
import jax
import jax.numpy as jnp
from jax.experimental import pallas as pl
from jax.experimental.pallas import tpu as pltpu

N = 10000
E = 320000
HID = 256
OUT = 256
CHUNK = 512


def _mm_kernel(x_ref, w_ref, o_ref):
    o_ref[...] = jnp.dot(x_ref[...], w_ref[...], preferred_element_type=jnp.float32)


def _mm(x, w):
    return pl.pallas_call(
        _mm_kernel,
        out_shape=jax.ShapeDtypeStruct((x.shape[0], w.shape[1]), jnp.float32),
    )(x, w)


def _agg1_kernel(src_ref, dst_ref, al_ref, h_ref, o_ref):
    step = pl.program_id(0)

    @pl.when(step == 0)
    def _init():
        o_ref[...] = jnp.zeros_like(o_ref)

    head_mask = jax.lax.broadcasted_iota(jnp.int32, (1, 2 * HID), 1) < HID

    def body(jj, carry):
        base = jj * 8
        msgs = []
        for u in range(8):
            j = base + u
            s = src_ref[j]
            a0 = al_ref[j, 0]
            a1 = al_ref[j, 1]
            row = h_ref[pl.ds(s, 1), :]
            mult = jnp.where(head_mask, a0, a1)
            msgs.append(row * mult)
        for u in range(8):
            j = base + u
            dn = dst_ref[j]
            o_ref[pl.ds(dn, 1), :] = o_ref[pl.ds(dn, 1), :] + msgs[u]
        return carry

    jax.lax.fori_loop(0, CHUNK // 8, body, 0)


def _agg1(src_s, dst_s, alpha_s, h):
    return pl.pallas_call(
        _agg1_kernel,
        grid=(E // CHUNK,),
        in_specs=[
            pl.BlockSpec((CHUNK,), lambda i: (i,), memory_space=pltpu.SMEM),
            pl.BlockSpec((CHUNK,), lambda i: (i,), memory_space=pltpu.SMEM),
            pl.BlockSpec((CHUNK, 2), lambda i: (i, 0), memory_space=pltpu.SMEM),
            pl.BlockSpec((N, 2 * HID), lambda i: (0, 0)),
        ],
        out_specs=pl.BlockSpec((N, 2 * HID), lambda i: (0, 0)),
        out_shape=jax.ShapeDtypeStruct((N, 2 * HID), jnp.float32),
    )(src_s, dst_s, alpha_s, h)


def _gat1(x, edge_index, W, att_src, att_dst, bias):
    src = edge_index[0]
    dst = edge_index[1]
    n = x.shape[0]
    h = _mm(x, W)
    h3 = h.reshape(n, 2, HID)
    a_src = jnp.sum(h3 * att_src[None, :, :], axis=-1)
    a_dst = jnp.sum(h3 * att_dst[None, :, :], axis=-1)
    alpha = a_src[src] + a_dst[dst]
    alpha = jax.nn.leaky_relu(alpha, negative_slope=0.2)
    amax = jax.ops.segment_max(alpha, dst, num_segments=n)
    alpha = jnp.exp(alpha - amax[dst])
    denom = jax.ops.segment_sum(alpha, dst, num_segments=n)
    alpha = alpha / (denom[dst] + 1e-16)
    perm = jnp.argsort(dst, stable=True)
    out = _agg1(src[perm], dst[perm], alpha[perm], h)
    return out + bias


def _gat2(x, edge_index, W, att_src, att_dst, bias):
    src = edge_index[0]
    dst = edge_index[1]
    n = x.shape[0]
    h = _mm(x, W).reshape(n, 1, OUT)
    a_src = jnp.sum(h * att_src[None, :, :], axis=-1)
    a_dst = jnp.sum(h * att_dst[None, :, :], axis=-1)
    alpha = a_src[src] + a_dst[dst]
    alpha = jax.nn.leaky_relu(alpha, negative_slope=0.2)
    amax = jax.ops.segment_max(alpha, dst, num_segments=n)
    alpha = jnp.exp(alpha - amax[dst])
    denom = jax.ops.segment_sum(alpha, dst, num_segments=n)
    alpha = alpha / (denom[dst] + 1e-16)
    msg = h[src] * alpha[:, :, None]
    out = jax.ops.segment_sum(msg, dst, num_segments=n)
    out = out.mean(axis=1)
    return out + bias


def _batchnorm(x, gamma, beta, eps=1e-5):
    mean = x.mean(axis=0)
    var = x.var(axis=0)
    return (x - mean) / jnp.sqrt(var + eps) * gamma + beta


def kernel(x, edge_index, W1, att_src1, att_dst1, b1, bn1_g, bn1_b,
           W2, att_src2, att_dst2, b2, bn2_g, bn2_b):
    h = _gat1(x, edge_index, W1, att_src1, att_dst1, b1)
    h = _batchnorm(h, bn1_g, bn1_b)
    h = jax.nn.relu(h)
    h = _gat2(h, edge_index, W2, att_src2, att_dst2, b2)
    h = _batchnorm(h, bn2_g, bn2_b)
    return h.mean(axis=0, keepdims=True)
